# baseline (speedup 1.0000x reference)
import functools
import jax
import jax.numpy as jnp
from jax import lax
from jax.experimental import pallas as pl
from jax.experimental.pallas import tpu as pltpu
from jax.experimental.pallas import tpu_sc as plsc

VOCAB = 100000
EMBED_DIM = 128
IMG_DIM = 512
OUT_DIM = 1000
OUT_PAD = 1024
BATCH = 4096
HIST = 50

NC, NS, L = 2, 16, 16
NW = NC * NS
B_PER_W = BATCH // NW
N_CHUNKS = B_PER_W
N_COL = EMBED_DIM // L
R_UNROLL = 5


def _embbag_body(table_hbm, wf_hbm, out_hbm, idx_v, rows_a, rows_b, acc_v,
                 sem_a, sem_b, sem_idx):
    wid = lax.axis_index("s") * NC + lax.axis_index("c")
    row_base = wid * B_PER_W

    pltpu.async_copy(
        wf_hbm.at[pl.ds(row_base, B_PER_W)], idx_v, sem_idx
    ).wait()

    def gathers(c, buf, sem):
        return [
            pltpu.make_async_copy(
                table_hbm.at[idx_v.at[pl.ds(c, 1)]],
                buf,
                sem,
            )
        ]

    def start(c, buf, sem):
        for g in gathers(c, buf, sem):
            g.start()

    def wait(c, buf, sem):
        for g in gathers(c, buf, sem):
            g.wait()

    def process(buf, c):
        def rbody(t, acc):
            for dr in range(R_UNROLL):
                acc = tuple(
                    acc[l] + buf[0, t * R_UNROLL + dr, pl.ds(L * l, L)]
                    for l in range(N_COL)
                )
            return acc

        acc = lax.fori_loop(
            0, HIST // R_UNROLL, rbody,
            tuple(jnp.zeros((L,), jnp.float32) for _ in range(N_COL)),
        )
        for l in range(N_COL):
            acc_v[c, pl.ds(L * l, L)] = acc[l]

    start(0, rows_a, sem_a)
    start(1, rows_b, sem_b)

    def chunk2(cc, _):
        c0 = 2 * cc
        c1 = 2 * cc + 1
        wait(c0, rows_a, sem_a)
        process(rows_a, c0)

        @pl.when(c0 + 2 < N_CHUNKS)
        def _():
            start(c0 + 2, rows_a, sem_a)

        wait(c1, rows_b, sem_b)
        process(rows_b, c1)

        @pl.when(c1 + 2 < N_CHUNKS)
        def _():
            start(c1 + 2, rows_b, sem_b)

        return 0

    lax.fori_loop(0, N_CHUNKS // 2, chunk2, 0)
    pltpu.sync_copy(acc_v, out_hbm.at[pl.ds(row_base, B_PER_W)])


@functools.cache
def _embbag():
    return pl.kernel(
        _embbag_body,
        out_type=jax.ShapeDtypeStruct((BATCH, EMBED_DIM), jnp.float32),
        mesh=plsc.VectorSubcoreMesh(
            core_axis_name="c", subcore_axis_name="s", num_cores=NC, num_subcores=NS
        ),
        scratch_types=[
            pltpu.VMEM((B_PER_W, HIST), jnp.int32),
            pltpu.VMEM((1, HIST, EMBED_DIM), jnp.float32),
            pltpu.VMEM((1, HIST, EMBED_DIM), jnp.float32),
            pltpu.VMEM((B_PER_W, EMBED_DIM), jnp.float32),
            pltpu.SemaphoreType.DMA,
            pltpu.SemaphoreType.DMA,
            pltpu.SemaphoreType.DMA,
        ],
    )


def _dense_softmax_body(emb_ref, img_ref, wt_ref, b_ref, out_ref):
    logits = (
        jnp.dot(emb_ref[...], wt_ref[:EMBED_DIM, :],
                preferred_element_type=jnp.float32)
        + jnp.dot(img_ref[...], wt_ref[EMBED_DIM:, :],
                  preferred_element_type=jnp.float32)
        + b_ref[...]
    )
    m = jnp.max(logits, axis=1, keepdims=True)
    e = jnp.exp(logits - m)
    out_ref[...] = e / jnp.sum(e, axis=1, keepdims=True)


def _dense_softmax(emb, img, wt, bp):
    BM = 512
    return pl.pallas_call(
        _dense_softmax_body,
        grid=(BATCH // BM,),
        in_specs=[
            pl.BlockSpec((BM, EMBED_DIM), lambda i: (i, 0)),
            pl.BlockSpec((BM, IMG_DIM), lambda i: (i, 0)),
            pl.BlockSpec((EMBED_DIM + IMG_DIM, OUT_PAD), lambda i: (0, 0)),
            pl.BlockSpec((1, OUT_PAD), lambda i: (0, 0)),
        ],
        out_specs=pl.BlockSpec((BM, OUT_PAD), lambda i: (i, 0)),
        out_shape=jax.ShapeDtypeStruct((BATCH, OUT_PAD), jnp.float32),
    )(emb, img, wt, bp)


@jax.jit
def kernel(word_features, image_features, emb_table, W, b):
    wf = word_features.astype(jnp.int32)
    emb = _embbag()(emb_table.reshape(1, VOCAB, EMBED_DIM), wf)
    scale = jnp.concatenate(
        [jnp.full((EMBED_DIM, 1), 1.0 / HIST, jnp.float32),
         jnp.ones((IMG_DIM, 1), jnp.float32)]
    )
    wt = jnp.pad(W.T * scale, ((0, 0), (0, OUT_PAD - OUT_DIM)))
    bp = jnp.pad(b, (0, OUT_PAD - OUT_DIM), constant_values=-1e30).reshape(1, OUT_PAD)
    out = _dense_softmax(emb, image_features, wt, bp)
    return out[:, :OUT_DIM]


# speedup vs baseline: 1.2762x; 1.2762x over previous
import functools
import jax
import jax.numpy as jnp
from jax import lax
from jax.experimental import pallas as pl
from jax.experimental.pallas import tpu as pltpu
from jax.experimental.pallas import tpu_sc as plsc

VOCAB = 100000
EMBED_DIM = 128
IMG_DIM = 512
OUT_DIM = 1000
OUT_PAD = 1024
BATCH = 4096
HIST = 50

NC, NS, L = 2, 16, 16
NW = NC * NS
B_PER_W = BATCH // NW
N_COL = EMBED_DIM // L

IDX_ROWS = BATCH * HIST // 128
ROWS_PER_W = IDX_ROWS // NW
STREAMS_PER_GROUP = 25
GROUPS = ROWS_PER_W // STREAMS_PER_GROUP
B_PER_GROUP = STREAMS_PER_GROUP * 128 // HIST


def _group_schedule():
    sched = []
    for k in range(STREAMS_PER_GROUP):
        s0, s1 = 128 * k, 128 * k + 128
        segs = []
        for r in range(s0 // HIST, (s1 - 1) // HIST + 1):
            a, b = max(s0, HIST * r), min(s1, HIST * r + HIST)
            segs.append((a - s0, b - a, r, a == HIST * r, b == HIST * r + HIST))
        sched.append(segs)
    return sched


_SCHED = _group_schedule()


def _embbag_body(table_hbm, wf_hbm, out_hbm, idx_v, rows_a, rows_b, acc_v,
                 sem_a, sem_b, sem_idx):
    wid = lax.axis_index("s") * NC + lax.axis_index("c")

    base = wid * ROWS_PER_W
    start8 = jnp.minimum((base // 8) * 8, IDX_ROWS - 64)
    local = base - start8
    pltpu.async_copy(
        wf_hbm.at[pl.ds(start8, 64)], idx_v, sem_idx
    ).wait()

    bufs = (rows_a, rows_b)
    sems = (sem_a, sem_b)

    def group(g, _):
        gbase = g * STREAMS_PER_GROUP
        acc_base = g * B_PER_GROUP

        def gather(k):
            return pltpu.make_async_copy(
                table_hbm.at[idx_v.at[pl.ds(local + gbase + k, 1)]],
                bufs[k % 2],
                sems[k % 2],
            )

        gather(0).start()
        gather(1).start()

        acc = None
        for k in range(STREAMS_PER_GROUP):
            buf = bufs[k % 2]
            gather(k).wait()
            if k + 2 < STREAMS_PER_GROUP:
                gather(k + 2).start()
            for (t0, n, r, is_start, is_end) in _SCHED[k]:
                if is_start:
                    acc = tuple(
                        jnp.zeros((L,), jnp.float32) for _ in range(N_COL)
                    )

                def seg_body(t, a, _t0=t0):
                    return tuple(
                        a[l] + buf[0, _t0 + t, pl.ds(L * l, L)]
                        for l in range(N_COL)
                    )

                acc = lax.fori_loop(0, n, seg_body, acc)
                if is_end:
                    for l in range(N_COL):
                        acc_v[acc_base + r, pl.ds(L * l, L)] = acc[l]
        return 0

    lax.fori_loop(0, GROUPS, group, 0)
    pltpu.sync_copy(acc_v, out_hbm.at[pl.ds(wid * B_PER_W, B_PER_W)])


@functools.cache
def _embbag():
    return pl.kernel(
        _embbag_body,
        out_type=jax.ShapeDtypeStruct((BATCH, EMBED_DIM), jnp.float32),
        mesh=plsc.VectorSubcoreMesh(
            core_axis_name="c", subcore_axis_name="s", num_cores=NC, num_subcores=NS
        ),
        scratch_types=[
            pltpu.VMEM((64, 128), jnp.int32),
            pltpu.VMEM((1, 128, EMBED_DIM), jnp.float32),
            pltpu.VMEM((1, 128, EMBED_DIM), jnp.float32),
            pltpu.VMEM((B_PER_W, EMBED_DIM), jnp.float32),
            pltpu.SemaphoreType.DMA,
            pltpu.SemaphoreType.DMA,
            pltpu.SemaphoreType.DMA,
        ],
    )


def _dense_softmax_body(emb_ref, img_ref, wt_ref, b_ref, out_ref):
    logits = (
        jnp.dot(emb_ref[...], wt_ref[:EMBED_DIM, :],
                preferred_element_type=jnp.float32)
        + jnp.dot(img_ref[...], wt_ref[EMBED_DIM:, :],
                  preferred_element_type=jnp.float32)
        + b_ref[...]
    )
    m = jnp.max(logits, axis=1, keepdims=True)
    e = jnp.exp(logits - m)
    out_ref[...] = e / jnp.sum(e, axis=1, keepdims=True)


def _dense_softmax(emb, img, wt, bp):
    BM = 512
    return pl.pallas_call(
        _dense_softmax_body,
        grid=(BATCH // BM,),
        in_specs=[
            pl.BlockSpec((BM, EMBED_DIM), lambda i: (i, 0)),
            pl.BlockSpec((BM, IMG_DIM), lambda i: (i, 0)),
            pl.BlockSpec((EMBED_DIM + IMG_DIM, OUT_PAD), lambda i: (0, 0)),
            pl.BlockSpec((1, OUT_PAD), lambda i: (0, 0)),
        ],
        out_specs=pl.BlockSpec((BM, OUT_PAD), lambda i: (i, 0)),
        out_shape=jax.ShapeDtypeStruct((BATCH, OUT_PAD), jnp.float32),
    )(emb, img, wt, bp)


@jax.jit
def kernel(word_features, image_features, emb_table, W, b):
    wf = word_features.astype(jnp.int32).reshape(IDX_ROWS, 128)
    wf = jnp.minimum(jnp.maximum(wf, 0), VOCAB - 1)
    emb = _embbag()(emb_table.reshape(1, VOCAB, EMBED_DIM), wf)
    scale = jnp.concatenate(
        [jnp.full((EMBED_DIM, 1), 1.0 / HIST, jnp.float32),
         jnp.ones((IMG_DIM, 1), jnp.float32)]
    )
    wt = jnp.pad(W.T * scale, ((0, 0), (0, OUT_PAD - OUT_DIM)))
    bp = jnp.pad(b, (0, OUT_PAD - OUT_DIM), constant_values=-1e30).reshape(1, OUT_PAD)
    out = _dense_softmax(emb, image_features, wt, bp)
    return out[:, :OUT_DIM]
